# baseline (speedup 1.0000x reference)
import functools

import jax
import jax.numpy as jnp
from jax import lax
from jax.experimental import pallas as pl
from jax.experimental.pallas import tpu as pltpu
from jax.experimental.pallas import tpu_sc as plsc

_FILL = -1000000000.0
_HIT = 10.0
_LANES = 16


def kernel(answer_token, anchor, action_dim):
    del anchor
    batch = answer_token.shape[0]
    adim = 100000
    answers = jnp.clip(answer_token.astype(jnp.int32), 0, action_dim - 1)

    num_cores = 2
    num_subcores = 16
    nw = num_cores * num_subcores
    rows_per_w = batch // nw
    ngroups = rows_per_w // 8
    chunk = 11008
    nfull = adim // chunk
    tail = adim - nfull * chunk

    mesh = plsc.VectorSubcoreMesh(core_axis_name="c", subcore_axis_name="s")

    @functools.partial(
        pl.kernel,
        mesh=mesh,
        out_type=jax.ShapeDtypeStruct((batch, adim), jnp.float32),
        scratch_types=[
            pltpu.VMEM((8, chunk), jnp.float32),
            pltpu.VMEM((8, tail), jnp.float32),
            pltpu.VMEM((rows_per_w,), jnp.int32),
            pltpu.SemaphoreType.DMA,
            pltpu.SemaphoreType.DMA,
        ],
    )
    def sc_fill(ans_hbm, out_hbm, buf_v, tail_v, ans_v, sem_a, sem_f):
        cid = lax.axis_index("c")
        sid = lax.axis_index("s")
        wid = sid * num_cores + cid
        base = wid * rows_per_w

        pltpu.async_copy(ans_hbm.at[pl.ds(base, rows_per_w)], ans_v, sem_a).wait()

        fill_vec = jnp.full((_LANES,), _FILL, jnp.float32)

        def fill_body(i, carry):
            j = i * _LANES
            buf_v[j // chunk, pl.ds(j % chunk, _LANES)] = fill_vec
            return carry

        lax.fori_loop(0, 8 * chunk // _LANES, fill_body, 0)

        def tail_body(i, carry):
            j = i * _LANES
            tail_v[j // tail, pl.ds(j % tail, _LANES)] = fill_vec
            return carry

        lax.fori_loop(0, 8 * tail // _LANES, tail_body, 0)

        copies = []
        for g in range(ngroups):
            r0 = base + g * 8
            for k in range(nfull):
                copies.append(
                    pltpu.async_copy(
                        buf_v,
                        out_hbm.at[pl.ds(r0, 8), pl.ds(k * chunk, chunk)],
                        sem_f,
                    )
                )
            copies.append(
                pltpu.async_copy(
                    tail_v,
                    out_hbm.at[pl.ds(r0, 8), pl.ds(nfull * chunk, tail)],
                    sem_f,
                )
            )
        for c in copies:
            c.wait()

    return sc_fill(answers)


# speedup vs baseline: 1.0976x; 1.0976x over previous
import functools

import jax
import jax.numpy as jnp
from jax.experimental import pallas as pl
from jax.experimental.pallas import tpu as pltpu

_FILL = -1000000000.0
_HIT = 10.0


def _fill_kernel(ans_ref, out_ref, *, block_cols: int):
    j = pl.program_id(1)
    rows, cols = out_ref.shape
    col0 = j * block_cols
    col_ids = col0 + jax.lax.broadcasted_iota(jnp.int32, (rows, cols), 1)
    ans = ans_ref[...]
    out_ref[...] = jnp.where(col_ids == ans, _HIT, _FILL).astype(jnp.float32)


def kernel(answer_token, anchor, action_dim):
    del anchor
    batch = answer_token.shape[0]
    action_dim_static = 100000
    answers = jnp.clip(answer_token.astype(jnp.int32), 0, action_dim - 1)
    answers = answers.reshape(batch, 1)

    block_rows = 32
    block_cols = action_dim_static
    grid = (pl.cdiv(batch, block_rows), pl.cdiv(action_dim_static, block_cols))

    return pl.pallas_call(
        functools.partial(_fill_kernel, block_cols=block_cols),
        grid=grid,
        in_specs=[pl.BlockSpec((block_rows, 1), lambda i, j: (i, 0))],
        out_specs=pl.BlockSpec((block_rows, block_cols), lambda i, j: (i, j)),
        out_shape=jax.ShapeDtypeStruct((batch, action_dim_static), jnp.float32),
        compiler_params=pltpu.CompilerParams(
            dimension_semantics=("parallel", "parallel"),
        ),
    )(answers)
